# baseline (speedup 1.0000x reference)
import jax
import jax.numpy as jnp
import numpy as np
from jax import lax
from jax.experimental import pallas as pl
from jax.experimental.pallas import tpu as pltpu
from jax.experimental.pallas import tpu_sc as plsc


def kernel(frames):
    C, T, H, W = frames.shape
    S = T // 4

    idx = np.linspace(0.0, T - 1, S).astype(np.int64)
    assert np.array_equal(idx, (np.arange(S) * (T - 1)) // (S - 1))
    sel = [(int(t), j) for j, t in enumerate(idx)]

    N = C * T
    NC, NS = 2, 16
    NW = NC * NS
    assert N % NW == 0
    RPW = N // NW

    HB = H // 2
    assert HB % 8 == 0
    NCH = RPW * 2
    SLOTS = 2

    mesh = plsc.VectorSubcoreMesh(
        core_axis_name="c", subcore_axis_name="s")

    def sc_body(x_hbm, fast_hbm, buf, insem, outsem):
        wid = lax.axis_index("s") * NC + lax.axis_index("c")

        def cinfo(k):
            r = wid * RPW + k // 2
            ch = lax.div(r, T)
            t = lax.rem(r, T)
            return ch, t, (k % 2) * HB

        def in_cp(k, s):
            ch, t, h0 = cinfo(k)
            return pltpu.make_async_copy(
                x_hbm.at[ch, t, pl.ds(h0, HB)], buf.at[s], insem.at[s])

        def out_cp(k, s):
            ch, t, h0 = cinfo(k)
            return pltpu.make_async_copy(
                buf.at[s], fast_hbm.at[ch, t, pl.ds(h0, HB)], outsem.at[s])

        for k in range(SLOTS):
            in_cp(k, k).start()
        for k in range(NCH):
            s = k % SLOTS
            if k >= 1:
                p = k - 1
                if p + SLOTS < NCH:
                    out_cp(p, p % SLOTS).wait()
                    in_cp(p + SLOTS, p % SLOTS).start()
            in_cp(k, s).wait()
            out_cp(k, s).start()
        for k in range(max(NCH - SLOTS, 0), NCH):
            out_cp(k, k % SLOTS).wait()

    sc_run = pl.kernel(
        sc_body,
        out_type=jax.ShapeDtypeStruct((C, T, H, W), frames.dtype),
        mesh=mesh,
        scratch_types=[
            pltpu.VMEM((SLOTS, HB, W), frames.dtype),
            pltpu.SemaphoreType.DMA((SLOTS,)),
            pltpu.SemaphoreType.DMA((SLOTS,)),
        ],
    )

    TSLOTS = 4
    work = [(c, t, j) for c in range(C) for (t, j) in sel]

    def tc_body(x_hbm, slow_hbm, buf, insem, outsem):
        def in_cp(i, s):
            c, t, _ = work[i]
            return pltpu.make_async_copy(
                x_hbm.at[c, t], buf.at[s], insem.at[s])

        def out_cp(i, s):
            c, _, j = work[i]
            return pltpu.make_async_copy(
                buf.at[s], slow_hbm.at[c, j], outsem.at[s])

        nw = len(work)
        for i in range(min(TSLOTS, nw)):
            in_cp(i, i).start()
        for i in range(nw):
            s = i % TSLOTS
            if i >= 1:
                p = i - 1
                if p + TSLOTS < nw:
                    out_cp(p, p % TSLOTS).wait()
                    in_cp(p + TSLOTS, p % TSLOTS).start()
            in_cp(i, s).wait()
            out_cp(i, s).start()
        for i in range(max(nw - TSLOTS, 0), nw):
            out_cp(i, i % TSLOTS).wait()

    slow = pl.pallas_call(
        tc_body,
        in_specs=[pl.BlockSpec(memory_space=pltpu.MemorySpace.HBM)],
        out_specs=pl.BlockSpec(memory_space=pltpu.MemorySpace.HBM),
        out_shape=jax.ShapeDtypeStruct((C, S, H, W), frames.dtype),
        scratch_shapes=[
            pltpu.VMEM((TSLOTS, H, W), frames.dtype),
            pltpu.SemaphoreType.DMA((TSLOTS,)),
            pltpu.SemaphoreType.DMA((TSLOTS,)),
        ],
    )(frames)

    fast = sc_run(frames)
    return (slow, fast)


# speedup vs baseline: 1.0722x; 1.0722x over previous
import jax
import jax.numpy as jnp
import numpy as np
from jax import lax
from jax.experimental import pallas as pl
from jax.experimental.pallas import tpu as pltpu
from jax.experimental.pallas import tpu_sc as plsc


def kernel(frames):
    C, T, H, W = frames.shape
    S = T // 4

    idx = np.linspace(0.0, T - 1, S).astype(np.int64)
    assert np.array_equal(idx, (np.arange(S) * (T - 1)) // (S - 1))

    NSEL = C * S
    NC, NS = 2, 16
    NW = NC * NS

    assert NSEL * 2 == 3 * NW
    HB = H // 2
    assert HB % 8 == 0

    mesh = plsc.VectorSubcoreMesh(
        core_axis_name="c", subcore_axis_name="s")

    def sc_body(x_hbm, slow_hbm, buf, insem, outsem):
        wid = lax.axis_index("s") * NC + lax.axis_index("c")
        two = wid < NS

        def finfo(f):
            ch = lax.div(f, S)
            jj = lax.rem(f, S)
            t = (jj * (T - 1)) // (S - 1)
            return ch, t, jj

        def in_cp(f, half, s):
            ch, t, _ = finfo(f)
            return pltpu.make_async_copy(
                x_hbm.at[ch, t, pl.ds(half * HB, HB)], buf.at[s],
                insem.at[s])

        def out_cp(f, half, s):
            ch, _, jj = finfo(f)
            return pltpu.make_async_copy(
                buf.at[s], slow_hbm.at[ch, jj, pl.ds(half * HB, HB)],
                outsem.at[s])

        f0 = jnp.where(two, 2 * wid, wid + NS)
        f1 = f0 + 1

        in_cp(f0, 0, 0).start()
        in_cp(f0, 1, 1).start()
        in_cp(f0, 0, 0).wait()
        out_cp(f0, 0, 0).start()
        in_cp(f0, 1, 1).wait()
        out_cp(f0, 1, 1).start()

        @pl.when(two)
        def _():
            out_cp(f0, 0, 0).wait()
            in_cp(f1, 0, 0).start()
            out_cp(f0, 1, 1).wait()
            in_cp(f1, 1, 1).start()
            in_cp(f1, 0, 0).wait()
            out_cp(f1, 0, 0).start()
            in_cp(f1, 1, 1).wait()
            out_cp(f1, 1, 1).start()
            out_cp(f1, 0, 0).wait()
            out_cp(f1, 1, 1).wait()

        @pl.when(jnp.logical_not(two))
        def _():
            out_cp(f0, 0, 0).wait()
            out_cp(f0, 1, 1).wait()

    sc_run = pl.kernel(
        sc_body,
        out_type=jax.ShapeDtypeStruct((C, S, H, W), frames.dtype),
        mesh=mesh,
        scratch_types=[
            pltpu.VMEM((2, HB, W), frames.dtype),
            pltpu.SemaphoreType.DMA((2,)),
            pltpu.SemaphoreType.DMA((2,)),
        ],
    )

    GF = 8
    assert T % GF == 0
    NG = C * (T // GF)
    TSLOTS = 4

    def tc_body(x_hbm, fast_hbm, buf, insem, outsem):
        def grp(g):
            return g // (T // GF), (g % (T // GF)) * GF

        def in_cp(g, s):
            ch, t0 = grp(g)
            return pltpu.make_async_copy(
                x_hbm.at[ch, pl.ds(t0, GF)], buf.at[s], insem.at[s])

        def out_cp(g, s):
            ch, t0 = grp(g)
            return pltpu.make_async_copy(
                buf.at[s], fast_hbm.at[ch, pl.ds(t0, GF)], outsem.at[s])

        for g in range(min(TSLOTS, NG)):
            in_cp(g, g).start()
        for g in range(NG):
            s = g % TSLOTS
            if g >= 1:
                p = g - 1
                if p + TSLOTS < NG:
                    out_cp(p, p % TSLOTS).wait()
                    in_cp(p + TSLOTS, p % TSLOTS).start()
            in_cp(g, s).wait()
            out_cp(g, s).start()
        for g in range(max(NG - TSLOTS, 0), NG):
            out_cp(g, g % TSLOTS).wait()

    fast = pl.pallas_call(
        tc_body,
        in_specs=[pl.BlockSpec(memory_space=pltpu.MemorySpace.HBM)],
        out_specs=pl.BlockSpec(memory_space=pltpu.MemorySpace.HBM),
        out_shape=jax.ShapeDtypeStruct((C, T, H, W), frames.dtype),
        scratch_shapes=[
            pltpu.VMEM((TSLOTS, GF, H, W), frames.dtype),
            pltpu.SemaphoreType.DMA((TSLOTS,)),
            pltpu.SemaphoreType.DMA((TSLOTS,)),
        ],
    )(frames)

    slow = sc_run(frames)
    return (slow, fast)
